# Initial kernel scaffold; baseline (speedup 1.0000x reference)
import jax
import jax.numpy as jnp
from jax.experimental import pallas as pl


def kernel(x, edge_index, edge_weight, W1, a_src1, a_dst1, a_edge1, We1, b1, W2, a_src2, a_dst2, a_edge2, We2, b2):
    raise NotImplementedError("write your pallas kernel here")


# speedup vs baseline: 20.7268x; 20.7268x over previous
import functools

import jax
import jax.numpy as jnp
from jax import lax
from jax.experimental import pallas as pl
from jax.experimental.pallas import tpu as pltpu
from jax.experimental.pallas import tpu_sc as plsc

N = 10000
E = 320000
F = 128
NC, NS = 2, 16
ROW = 80
LK = 16
NWIN = 10
WR = 25
FH = F // NC
NEG_SLOPE = 0.2
EPS = 1e-16

_GDN = lax.GatherDimensionNumbers(
    offset_dims=(), collapsed_slice_dims=(0,), start_index_map=(0,))


def _bcast16(vec, k):
    idx = jnp.full((LK, 1), k, jnp.int32)
    return lax.gather(vec, idx, _GDN, (1,),
                      mode=lax.GatherScatterMode.PROMISE_IN_BOUNDS)


def _tc_pre_body(x_ref, w_ref, asrc_ref, adst_ref, aedge_ref, we_ref, ew_ref,
                 xh_ref, as_ref, ad_ref, scal_ref):
    xh = jnp.dot(x_ref[...], w_ref[...], preferred_element_type=jnp.float32)
    xh_ref[0] = xh[:, :FH]
    xh_ref[1] = xh[:, FH:]
    as_ref[...] = jnp.sum(xh * asrc_ref[...], axis=1, keepdims=True)
    ad_ref[...] = jnp.sum(xh * adst_ref[...], axis=1, keepdims=True)
    c = jnp.sum(we_ref[...] * aedge_ref[...])
    mean_w = jnp.mean(ew_ref[...])
    ones = jnp.ones((1, LK), jnp.float32)
    scal_ref[0:1, :] = ones * c
    scal_ref[1:2, :] = ones * (c * mean_w)
    scal_ref[2:3, :] = ones * mean_w
    scal_ref[3:4, :] = ones * 0.0


def _loop_coef(as_ref, ad_ref, asum_ref, scal_ref):
    l = as_ref[...] + ad_ref[...] + scal_ref[1:2, 0:1]
    l = jnp.where(l > 0, l, l * NEG_SLOPE)
    p = jnp.exp(l)
    return p / (asum_ref[...] + EPS)


def _tc_mid_body(part_ref, asum_ref, as_ref, ad_ref, scal_ref, xh_ref, b_ref,
                 w2_ref, asrc2_ref, adst2_ref, aedge2_ref, we2_ref,
                 xh2_ref, as2_ref, ad2_ref, scal2_ref):
    coef = _loop_coef(as_ref, ad_ref, asum_ref, scal_ref)
    part = jnp.concatenate([part_ref[0], part_ref[1]], axis=1)
    xh = jnp.concatenate([xh_ref[0], xh_ref[1]], axis=1)
    h = part + coef * xh + b_ref[...]
    h = jnp.maximum(h, 0.0)
    xh2 = jnp.dot(h, w2_ref[...], preferred_element_type=jnp.float32)
    xh2_ref[0] = xh2[:, :FH]
    xh2_ref[1] = xh2[:, FH:]
    as2_ref[...] = jnp.sum(xh2 * asrc2_ref[...], axis=1, keepdims=True)
    ad2_ref[...] = jnp.sum(xh2 * adst2_ref[...], axis=1, keepdims=True)
    c2 = jnp.sum(we2_ref[...] * aedge2_ref[...])
    mean_w = scal_ref[2:3, :]
    ones = jnp.ones((1, LK), jnp.float32)
    scal2_ref[0:1, :] = ones * c2
    scal2_ref[1:2, :] = c2 * mean_w
    scal2_ref[2:3, :] = mean_w
    scal2_ref[3:4, :] = ones * 0.0


def _tc_post_body(part_ref, asum_ref, as_ref, ad_ref, scal_ref, xh_ref, b_ref,
                  out_ref):
    coef = _loop_coef(as_ref, ad_ref, asum_ref, scal_ref)
    part = jnp.concatenate([part_ref[0], part_ref[1]], axis=1)
    xh = jnp.concatenate([xh_ref[0], xh_ref[1]], axis=1)
    out_ref[...] = part + coef * xh + b_ref[...]


def _f32(shape):
    return jax.ShapeDtypeStruct(shape, jnp.float32)


_tc_pre = pl.pallas_call(
    _tc_pre_body,
    out_shape=(_f32((NC, N, FH)), _f32((N, 1)), _f32((N, 1)), _f32((4, LK))),
)

BN = 2000

_bs_split = pl.BlockSpec((NC, BN, FH), lambda i: (0, i, 0))
_bs_col = pl.BlockSpec((BN, 1), lambda i: (i, 0))
_bs_scal = pl.BlockSpec((4, LK), lambda i: (0, 0))
_bs_b = pl.BlockSpec((1, F), lambda i: (0, 0))
_bs_w = pl.BlockSpec((F, F), lambda i: (0, 0))

_tc_mid = pl.pallas_call(
    _tc_mid_body,
    grid=(N // BN,),
    in_specs=[_bs_split, _bs_col, _bs_col, _bs_col, _bs_scal, _bs_split,
              _bs_b, _bs_w, _bs_b, _bs_b, _bs_b, _bs_b],
    out_specs=(_bs_split, _bs_col, _bs_col, _bs_scal),
    out_shape=(_f32((NC, N, FH)), _f32((N, 1)), _f32((N, 1)), _f32((4, LK))),
)

_tc_post = pl.pallas_call(
    _tc_post_body,
    grid=(N // BN,),
    in_specs=[_bs_split, _bs_col, _bs_col, _bs_col, _bs_scal, _bs_split,
              _bs_b],
    out_specs=pl.BlockSpec((BN, F), lambda i: (i, 0)),
    out_shape=_f32((N, F)),
)


def _sc_body(xh_hbm, src_hbm, dst_hbm, ew_hbm, as_hbm, ad_hbm, scal_hbm,
             part_hbm, asum_hbm,
             src_w, dst_w, ew_w, as_v, ad_v, asum_v, coef_v, rows_v,
             z640_v, st640_v, scal_v, out_sh, asum_sh, sem):
    c = lax.axis_index("c")
    s = lax.axis_index("s")
    zf = jnp.zeros((LK,), jnp.float32)

    pltpu.sync_copy(scal_hbm, scal_v)

    def _zrow(r, carry):
        for j in range(FH // LK):
            rows_v[r, pl.ds(LK * j, LK)] = zf
        return carry
    lax.fori_loop(0, ROW, _zrow, 0)

    def _zz(i, carry):
        z640_v[pl.ds(LK * i, LK)] = zf
        return carry
    lax.fori_loop(0, 40, _zz, 0)

    base = s * 640
    tail_n = N - 640 * (NS - 1)

    def _zero_shared(nn):
        for t in range(nn // ROW):
            pltpu.sync_copy(rows_v, out_sh.at[pl.ds(base + ROW * t, ROW)])
        pltpu.sync_copy(z640_v.at[pl.ds(0, nn)], asum_sh.at[pl.ds(base, nn)])

    @pl.when(s < NS - 1)
    def _():
        _zero_shared(640)

    @pl.when(s == NS - 1)
    def _():
        _zero_shared(tail_n)

    plsc.subcore_barrier()

    c1v = scal_v[0, :]
    cm1v = scal_v[1, :]
    pltpu.sync_copy(as_hbm, as_v)
    pltpu.sync_copy(ad_hbm, ad_v)

    def _ph1_win(t, carry):
        pltpu.sync_copy(src_hbm.at[s, t], src_w)
        pltpu.sync_copy(dst_hbm.at[s, t], dst_w)
        pltpu.sync_copy(ew_hbm.at[s, t], ew_w)

        def _ph1_row(r, carry2):
            for k in range(ROW // LK):
                sl = pl.ds(LK * k, LK)
                s16 = src_w[r, sl]
                d16 = dst_w[r, sl]
                e16 = ew_w[r, sl]
                av = plsc.load_gather(as_v, [s16])
                bv = plsc.load_gather(ad_v, [d16])
                l = av + bv + c1v * e16
                l = jnp.where(l > 0, l, l * NEG_SLOPE)
                ew_w[r, sl] = jnp.exp(l)
            pltpu.sync_copy(ew_w.at[r], asum_sh.at[dst_w.at[r]], add=True)
            return carry2
        lax.fori_loop(0, WR, _ph1_row, 0)
        return carry
    lax.fori_loop(0, NWIN, _ph1_win, 0)

    plsc.subcore_barrier()

    def _self_loop(nn):
        def _sl_chunk(i, carry):
            sl = pl.ds(LK * i, LK)
            l = (as_v[pl.ds(base + LK * i, LK)]
                 + ad_v[pl.ds(base + LK * i, LK)] + cm1v)
            l = jnp.where(l > 0, l, l * NEG_SLOPE)
            z640_v[sl] = jnp.exp(l)
            return carry
        lax.fori_loop(0, nn // LK, _sl_chunk, 0)
        pltpu.sync_copy(asum_sh.at[pl.ds(base, nn)], st640_v.at[pl.ds(0, nn)])

        def _sl_add(i, carry):
            sl = pl.ds(LK * i, LK)
            st640_v[sl] = st640_v[sl] + z640_v[sl]
            return carry
        lax.fori_loop(0, nn // LK, _sl_add, 0)
        pltpu.sync_copy(st640_v.at[pl.ds(0, nn)], asum_sh.at[pl.ds(base, nn)])

    @pl.when(s < NS - 1)
    def _():
        _self_loop(640)

    @pl.when(s == NS - 1)
    def _():
        _self_loop(tail_n)

    plsc.subcore_barrier()

    pltpu.sync_copy(asum_sh, asum_v)

    @pl.when(jnp.logical_and(c == 0, s < NS - 1))
    def _():
        pltpu.sync_copy(asum_v.at[pl.ds(base, 640)],
                        asum_hbm.at[pl.ds(base, 640)])

    @pl.when(jnp.logical_and(c == 0, s == NS - 1))
    def _():
        pltpu.sync_copy(asum_v.at[pl.ds(base, tail_n)],
                        asum_hbm.at[pl.ds(base, tail_n)])

    def _ph2_win(t, carry):
        pltpu.sync_copy(src_hbm.at[s, t], src_w)
        pltpu.sync_copy(dst_hbm.at[s, t], dst_w)
        pltpu.sync_copy(ew_hbm.at[s, t], ew_w)

        def _ph2_row(r, carry2):
            pltpu.async_copy(xh_hbm.at[c].at[src_w.at[r]], rows_v, sem).wait()
            for k in range(ROW // LK):
                sl = pl.ds(LK * k, LK)
                s16 = src_w[r, sl]
                d16 = dst_w[r, sl]
                e16 = ew_w[r, sl]
                av = plsc.load_gather(as_v, [s16])
                bv = plsc.load_gather(ad_v, [d16])
                l = av + bv + c1v * e16
                l = jnp.where(l > 0, l, l * NEG_SLOPE)
                p16 = jnp.exp(l)
                sa = plsc.load_gather(asum_v, [d16])
                coef_v[sl] = p16 / (sa + EPS)
            for k16 in range(ROW // LK):
                coefc = coef_v[pl.ds(LK * k16, LK)]
                for kk in range(LK):
                    cb = _bcast16(coefc, kk)
                    k = LK * k16 + kk
                    for j8 in range(FH // LK):
                        sl = pl.ds(LK * j8, LK)
                        rows_v[k, sl] = rows_v[k, sl] * cb
            pltpu.sync_copy(rows_v, out_sh.at[dst_w.at[r]], add=True)
            return carry2
        lax.fori_loop(0, WR, _ph2_row, 0)
        return carry
    lax.fori_loop(0, NWIN, _ph2_win, 0)

    plsc.subcore_barrier()

    def _wout(nn):
        for t in range(nn // ROW):
            pltpu.sync_copy(out_sh.at[pl.ds(base + ROW * t, ROW)], rows_v)
            pltpu.sync_copy(rows_v, part_hbm.at[c, pl.ds(base + ROW * t, ROW)])

    @pl.when(s < NS - 1)
    def _():
        _wout(640)

    @pl.when(s == NS - 1)
    def _():
        _wout(tail_n)


_sc_edge_pass = pl.kernel(
    _sc_body,
    out_type=(_f32((NC, N, FH)), _f32((N,))),
    mesh=plsc.VectorSubcoreMesh(core_axis_name="c", subcore_axis_name="s"),
    compiler_params=pltpu.CompilerParams(
        needs_layout_passes=False, use_tc_tiling_on_sc=False),
    scratch_types=[
        pltpu.VMEM((WR, ROW), jnp.int32),
        pltpu.VMEM((WR, ROW), jnp.int32),
        pltpu.VMEM((WR, ROW), jnp.float32),
        pltpu.VMEM((N,), jnp.float32),
        pltpu.VMEM((N,), jnp.float32),
        pltpu.VMEM((N,), jnp.float32),
        pltpu.VMEM((ROW,), jnp.float32),
        pltpu.VMEM((ROW, FH), jnp.float32),
        pltpu.VMEM((640,), jnp.float32),
        pltpu.VMEM((640,), jnp.float32),
        pltpu.VMEM((4, LK), jnp.float32),
        pltpu.VMEM_SHARED((N, FH), jnp.float32),
        pltpu.VMEM_SHARED((N,), jnp.float32),
        pltpu.SemaphoreType.DMA,
    ],
)


def kernel(x, edge_index, edge_weight, W1, a_src1, a_dst1, a_edge1, We1, b1,
           W2, a_src2, a_dst2, a_edge2, We2, b2):
    src2d = edge_index[0].reshape(NS, NWIN, WR, ROW)
    dst2d = edge_index[1].reshape(NS, NWIN, WR, ROW)
    ew2d = edge_weight.reshape(NS, NWIN, WR, ROW)

    xh1, as1, ad1, scal1 = _tc_pre(
        x, W1, a_src1.reshape(1, F), a_dst1.reshape(1, F),
        a_edge1.reshape(1, F), We1.reshape(1, F), edge_weight.reshape(-1, F))
    part1, asum1 = _sc_edge_pass(
        xh1, src2d, dst2d, ew2d, as1.reshape(N), ad1.reshape(N), scal1)
    xh2, as2, ad2, scal2 = _tc_mid(
        part1, asum1.reshape(N, 1), as1, ad1, scal1, xh1, b1.reshape(1, F),
        W2, a_src2.reshape(1, F), a_dst2.reshape(1, F),
        a_edge2.reshape(1, F), We2.reshape(1, F))
    part2, asum2 = _sc_edge_pass(
        xh2, src2d, dst2d, ew2d, as2.reshape(N), ad2.reshape(N), scal2)
    return _tc_post(
        part2, asum2.reshape(N, 1), as2, ad2, scal2, xh2, b2.reshape(1, F))


# speedup vs baseline: 27.3611x; 1.3201x over previous
import functools

import jax
import jax.numpy as jnp
from jax import lax
from jax.experimental import pallas as pl
from jax.experimental.pallas import tpu as pltpu
from jax.experimental.pallas import tpu_sc as plsc

N = 10000
E = 320000
F = 128
NC, NS = 2, 16
ROW = 80
LK = 16
NWIN = 10
WR = 25
FH = F // NC
NEG_SLOPE = 0.2
EPS = 1e-16

_GDN = lax.GatherDimensionNumbers(
    offset_dims=(), collapsed_slice_dims=(0,), start_index_map=(0,))


def _bcast16(vec, k):
    idx = jnp.full((LK, 1), k, jnp.int32)
    return lax.gather(vec, idx, _GDN, (1,),
                      mode=lax.GatherScatterMode.PROMISE_IN_BOUNDS)


def _tc_pre_body(x_ref, w_ref, asrc_ref, adst_ref, aedge_ref, we_ref, ew_ref,
                 xh_ref, as_ref, ad_ref, scal_ref):
    xh = jnp.dot(x_ref[...], w_ref[...], preferred_element_type=jnp.float32)
    xh_ref[0] = xh[:, :FH]
    xh_ref[1] = xh[:, FH:]
    as_ref[...] = jnp.sum(xh * asrc_ref[...], axis=1, keepdims=True)
    ad_ref[...] = jnp.sum(xh * adst_ref[...], axis=1, keepdims=True)
    c = jnp.sum(we_ref[...] * aedge_ref[...])
    mean_w = jnp.mean(ew_ref[...])
    ones = jnp.ones((1, LK), jnp.float32)
    scal_ref[0:1, :] = ones * c
    scal_ref[1:2, :] = ones * (c * mean_w)
    scal_ref[2:3, :] = ones * mean_w
    scal_ref[3:4, :] = ones * 0.0


def _loop_coef(as_ref, ad_ref, asum_ref, scal_ref):
    l = as_ref[...] + ad_ref[...] + scal_ref[1:2, 0:1]
    l = jnp.where(l > 0, l, l * NEG_SLOPE)
    p = jnp.exp(l)
    return p / (asum_ref[...] + EPS)


def _tc_mid_body(part_ref, asum_ref, as_ref, ad_ref, scal_ref, xh_ref, b_ref,
                 w2_ref, asrc2_ref, adst2_ref, aedge2_ref, we2_ref,
                 xh2_ref, as2_ref, ad2_ref, scal2_ref):
    coef = _loop_coef(as_ref, ad_ref, asum_ref, scal_ref)
    part = jnp.concatenate([part_ref[0], part_ref[1]], axis=1)
    xh = jnp.concatenate([xh_ref[0], xh_ref[1]], axis=1)
    h = part + coef * xh + b_ref[...]
    h = jnp.maximum(h, 0.0)
    xh2 = jnp.dot(h, w2_ref[...], preferred_element_type=jnp.float32)
    xh2_ref[0] = xh2[:, :FH]
    xh2_ref[1] = xh2[:, FH:]
    as2_ref[...] = jnp.sum(xh2 * asrc2_ref[...], axis=1, keepdims=True)
    ad2_ref[...] = jnp.sum(xh2 * adst2_ref[...], axis=1, keepdims=True)
    c2 = jnp.sum(we2_ref[...] * aedge2_ref[...])
    mean_w = scal_ref[2:3, :]
    ones = jnp.ones((1, LK), jnp.float32)
    scal2_ref[0:1, :] = ones * c2
    scal2_ref[1:2, :] = c2 * mean_w
    scal2_ref[2:3, :] = mean_w
    scal2_ref[3:4, :] = ones * 0.0


def _tc_post_body(part_ref, asum_ref, as_ref, ad_ref, scal_ref, xh_ref, b_ref,
                  out_ref):
    coef = _loop_coef(as_ref, ad_ref, asum_ref, scal_ref)
    part = jnp.concatenate([part_ref[0], part_ref[1]], axis=1)
    xh = jnp.concatenate([xh_ref[0], xh_ref[1]], axis=1)
    out_ref[...] = part + coef * xh + b_ref[...]


def _f32(shape):
    return jax.ShapeDtypeStruct(shape, jnp.float32)


_tc_pre = pl.pallas_call(
    _tc_pre_body,
    out_shape=(_f32((NC, N, FH)), _f32((N, 1)), _f32((N, 1)), _f32((4, LK))),
)

BN = 2000

_bs_split = pl.BlockSpec((NC, BN, FH), lambda i: (0, i, 0))
_bs_col = pl.BlockSpec((BN, 1), lambda i: (i, 0))
_bs_scal = pl.BlockSpec((4, LK), lambda i: (0, 0))
_bs_b = pl.BlockSpec((1, F), lambda i: (0, 0))
_bs_w = pl.BlockSpec((F, F), lambda i: (0, 0))

_tc_mid = pl.pallas_call(
    _tc_mid_body,
    grid=(N // BN,),
    in_specs=[_bs_split, _bs_col, _bs_col, _bs_col, _bs_scal, _bs_split,
              _bs_b, _bs_w, _bs_b, _bs_b, _bs_b, _bs_b],
    out_specs=(_bs_split, _bs_col, _bs_col, _bs_scal),
    out_shape=(_f32((NC, N, FH)), _f32((N, 1)), _f32((N, 1)), _f32((4, LK))),
)

_tc_post = pl.pallas_call(
    _tc_post_body,
    grid=(N // BN,),
    in_specs=[_bs_split, _bs_col, _bs_col, _bs_col, _bs_scal, _bs_split,
              _bs_b],
    out_specs=pl.BlockSpec((BN, F), lambda i: (i, 0)),
    out_shape=_f32((N, F)),
)


def _sc_body(xh_hbm, src_hbm, dst_hbm, ew_hbm, as_hbm, ad_hbm, scal_hbm,
             part_hbm, asum_hbm,
             src_w, dst_w, ew_w, as_v, ad_v, asum_v, rows_v,
             z640_v, st640_v, scal_v, out_sh, asum_sh, sem):
    c = lax.axis_index("c")
    s = lax.axis_index("s")
    zf = jnp.zeros((LK,), jnp.float32)

    pltpu.sync_copy(scal_hbm, scal_v)

    def _zrow(r, carry):
        for bb in range(2):
            for j in range(FH // LK):
                rows_v[bb, r, pl.ds(LK * j, LK)] = zf
        return carry
    lax.fori_loop(0, ROW, _zrow, 0)

    def _zz(i, carry):
        z640_v[pl.ds(LK * i, LK)] = zf
        return carry
    lax.fori_loop(0, 40, _zz, 0)

    base = s * 640
    tail_n = N - 640 * (NS - 1)

    def _zero_shared(nn):
        for t in range(nn // ROW):
            pltpu.sync_copy(rows_v.at[0], out_sh.at[pl.ds(base + ROW * t, ROW)])
        pltpu.sync_copy(z640_v.at[pl.ds(0, nn)], asum_sh.at[pl.ds(base, nn)])

    @pl.when(s < NS - 1)
    def _():
        _zero_shared(640)

    @pl.when(s == NS - 1)
    def _():
        _zero_shared(tail_n)

    plsc.subcore_barrier()

    c1v = scal_v[0, :]
    cm1v = scal_v[1, :]
    pltpu.sync_copy(as_hbm, as_v)
    pltpu.sync_copy(ad_hbm, ad_v)

    def _ph1_win(t, carry):
        pltpu.sync_copy(src_hbm.at[s, t], src_w)
        pltpu.sync_copy(dst_hbm.at[s, t], dst_w)
        pltpu.sync_copy(ew_hbm.at[s, t], ew_w)

        def _ph1_row(r, carry2):
            for k in range(ROW // LK):
                sl = pl.ds(LK * k, LK)
                s16 = src_w[r, sl]
                d16 = dst_w[r, sl]
                e16 = ew_w[r, sl]
                av = plsc.load_gather(as_v, [s16])
                bv = plsc.load_gather(ad_v, [d16])
                l = av + bv + c1v * e16
                l = jnp.where(l > 0, l, l * NEG_SLOPE)
                ew_w[r, sl] = jnp.exp(l)
            pltpu.sync_copy(ew_w.at[r], asum_sh.at[dst_w.at[r]], add=True)
            return carry2
        lax.fori_loop(0, WR, _ph1_row, 0)
        return carry
    lax.fori_loop(0, NWIN, _ph1_win, 0)

    plsc.subcore_barrier()

    def _self_loop(nn):
        def _sl_chunk(i, carry):
            sl = pl.ds(LK * i, LK)
            l = (as_v[pl.ds(base + LK * i, LK)]
                 + ad_v[pl.ds(base + LK * i, LK)] + cm1v)
            l = jnp.where(l > 0, l, l * NEG_SLOPE)
            z640_v[sl] = jnp.exp(l)
            return carry
        lax.fori_loop(0, nn // LK, _sl_chunk, 0)
        pltpu.sync_copy(asum_sh.at[pl.ds(base, nn)], st640_v.at[pl.ds(0, nn)])

        def _sl_add(i, carry):
            sl = pl.ds(LK * i, LK)
            st640_v[sl] = st640_v[sl] + z640_v[sl]
            return carry
        lax.fori_loop(0, nn // LK, _sl_add, 0)
        pltpu.sync_copy(st640_v.at[pl.ds(0, nn)], asum_sh.at[pl.ds(base, nn)])

    @pl.when(s < NS - 1)
    def _():
        _self_loop(640)

    @pl.when(s == NS - 1)
    def _():
        _self_loop(tail_n)

    plsc.subcore_barrier()

    pltpu.sync_copy(asum_sh, asum_v)

    @pl.when(jnp.logical_and(c == 0, s < NS - 1))
    def _():
        pltpu.sync_copy(asum_v.at[pl.ds(base, 640)],
                        asum_hbm.at[pl.ds(base, 640)])

    @pl.when(jnp.logical_and(c == 0, s == NS - 1))
    def _():
        pltpu.sync_copy(asum_v.at[pl.ds(base, tail_n)],
                        asum_hbm.at[pl.ds(base, tail_n)])

    def _gat(r, b):
        return pltpu.make_async_copy(
            xh_hbm.at[c].at[src_w.at[r]], rows_v.at[b], sem.at[b])

    def _sca(r, b):
        return pltpu.make_async_copy(
            rows_v.at[b], out_sh.at[dst_w.at[r]], sem.at[2 + b])

    def _ph2_win(t, carry):
        pltpu.sync_copy(src_hbm.at[s, t], src_w)
        pltpu.sync_copy(dst_hbm.at[s, t], dst_w)
        pltpu.sync_copy(ew_hbm.at[s, t], ew_w)
        _gat(0, 0).start()

        def _ph2_row(r, carry2):
            b = r % 2
            _gat(r, b).wait()

            @pl.when(r >= 1)
            def _():
                _sca(r - 1, 1 - b).wait()

            @pl.when(r < WR - 1)
            def _():
                _gat(r + 1, 1 - b).start()

            for k16 in range(ROW // LK):
                sl = pl.ds(LK * k16, LK)
                s16 = src_w[r, sl]
                d16 = dst_w[r, sl]
                e16 = ew_w[r, sl]
                av = plsc.load_gather(as_v, [s16])
                bv = plsc.load_gather(ad_v, [d16])
                l = av + bv + c1v * e16
                l = jnp.where(l > 0, l, l * NEG_SLOPE)
                p16 = jnp.exp(l)
                sa = plsc.load_gather(asum_v, [d16])
                coefc = p16 / (sa + EPS)
                for kk in range(LK):
                    cb = _bcast16(coefc, kk)
                    k = LK * k16 + kk
                    for j8 in range(FH // LK):
                        sl2 = pl.ds(LK * j8, LK)
                        rows_v[b, k, sl2] = rows_v[b, k, sl2] * cb
            _sca(r, b).start(add=True)
            return carry2
        lax.fori_loop(0, WR, _ph2_row, 0)
        _sca(WR - 1, (WR - 1) % 2).wait()
        return carry
    lax.fori_loop(0, NWIN, _ph2_win, 0)

    plsc.subcore_barrier()

    def _wout(nn):
        for t in range(nn // ROW):
            bb = t % 2
            pltpu.sync_copy(out_sh.at[pl.ds(base + ROW * t, ROW)], rows_v.at[bb])
            pltpu.sync_copy(rows_v.at[bb], part_hbm.at[c, pl.ds(base + ROW * t, ROW)])

    @pl.when(s < NS - 1)
    def _():
        _wout(640)

    @pl.when(s == NS - 1)
    def _():
        _wout(tail_n)


_sc_edge_pass = pl.kernel(
    _sc_body,
    out_type=(_f32((NC, N, FH)), _f32((N,))),
    mesh=plsc.VectorSubcoreMesh(core_axis_name="c", subcore_axis_name="s"),
    compiler_params=pltpu.CompilerParams(
        needs_layout_passes=False, use_tc_tiling_on_sc=False),
    scratch_types=[
        pltpu.VMEM((WR, ROW), jnp.int32),
        pltpu.VMEM((WR, ROW), jnp.int32),
        pltpu.VMEM((WR, ROW), jnp.float32),
        pltpu.VMEM((N,), jnp.float32),
        pltpu.VMEM((N,), jnp.float32),
        pltpu.VMEM((N,), jnp.float32),
        pltpu.VMEM((2, ROW, FH), jnp.float32),
        pltpu.VMEM((640,), jnp.float32),
        pltpu.VMEM((640,), jnp.float32),
        pltpu.VMEM((4, LK), jnp.float32),
        pltpu.VMEM_SHARED((N, FH), jnp.float32),
        pltpu.VMEM_SHARED((N,), jnp.float32),
        pltpu.SemaphoreType.DMA((4,)),
    ],
)


def kernel(x, edge_index, edge_weight, W1, a_src1, a_dst1, a_edge1, We1, b1,
           W2, a_src2, a_dst2, a_edge2, We2, b2):
    src2d = edge_index[0].reshape(NS, NWIN, WR, ROW)
    dst2d = edge_index[1].reshape(NS, NWIN, WR, ROW)
    ew2d = edge_weight.reshape(NS, NWIN, WR, ROW)

    xh1, as1, ad1, scal1 = _tc_pre(
        x, W1, a_src1.reshape(1, F), a_dst1.reshape(1, F),
        a_edge1.reshape(1, F), We1.reshape(1, F), edge_weight.reshape(-1, F))
    part1, asum1 = _sc_edge_pass(
        xh1, src2d, dst2d, ew2d, as1.reshape(N), ad1.reshape(N), scal1)
    xh2, as2, ad2, scal2 = _tc_mid(
        part1, asum1.reshape(N, 1), as1, ad1, scal1, xh1, b1.reshape(1, F),
        W2, a_src2.reshape(1, F), a_dst2.reshape(1, F),
        a_edge2.reshape(1, F), We2.reshape(1, F))
    part2, asum2 = _sc_edge_pass(
        xh2, src2d, dst2d, ew2d, as2.reshape(N), ad2.reshape(N), scal2)
    return _tc_post(
        part2, asum2.reshape(N, 1), as2, ad2, scal2, xh2, b2.reshape(1, F))


# speedup vs baseline: 28.9514x; 1.0581x over previous
import functools

import jax
import jax.numpy as jnp
from jax import lax
from jax.experimental import pallas as pl
from jax.experimental.pallas import tpu as pltpu
from jax.experimental.pallas import tpu_sc as plsc

N = 10000
E = 320000
F = 128
NC, NS = 2, 16
ROW = 80
LK = 16
NWIN = 10
WR = 25
FH = F // NC
NEG_SLOPE = 0.2
EPS = 1e-16

_GDN = lax.GatherDimensionNumbers(
    offset_dims=(), collapsed_slice_dims=(0,), start_index_map=(0,))


def _bcast16(vec, k):
    idx = jnp.full((LK, 1), k, jnp.int32)
    return lax.gather(vec, idx, _GDN, (1,),
                      mode=lax.GatherScatterMode.PROMISE_IN_BOUNDS)


def _tc_pre_body(x_ref, w_ref, asrc_ref, adst_ref, aedge_ref, we_ref, ew_ref,
                 xh_ref, as_ref, ad_ref, scal_ref):
    xh = jnp.dot(x_ref[...], w_ref[...], preferred_element_type=jnp.float32)
    xh_ref[0] = xh[:, :FH]
    xh_ref[1] = xh[:, FH:]
    as_ref[...] = jnp.sum(xh * asrc_ref[...], axis=1, keepdims=True)
    ad_ref[...] = jnp.sum(xh * adst_ref[...], axis=1, keepdims=True)
    c = jnp.sum(we_ref[...] * aedge_ref[...])
    mean_w = jnp.mean(ew_ref[...])
    ones = jnp.ones((1, LK), jnp.float32)
    scal_ref[0:1, :] = ones * c
    scal_ref[1:2, :] = ones * (c * mean_w)
    scal_ref[2:3, :] = ones * mean_w
    scal_ref[3:4, :] = ones * 0.0


def _loop_coef(as_ref, ad_ref, asum_ref, scal_ref):
    l = as_ref[...] + ad_ref[...] + scal_ref[1:2, 0:1]
    l = jnp.where(l > 0, l, l * NEG_SLOPE)
    p = jnp.exp(l)
    return p / (asum_ref[...] + EPS)


def _tc_mid_body(part_ref, asum_ref, as_ref, ad_ref, scal_ref, xh_ref, b_ref,
                 w2_ref, asrc2_ref, adst2_ref, aedge2_ref, we2_ref,
                 xh2_ref, as2_ref, ad2_ref, scal2_ref):
    coef = _loop_coef(as_ref, ad_ref, asum_ref, scal_ref)
    part = jnp.concatenate([part_ref[0], part_ref[1]], axis=1)
    xh = jnp.concatenate([xh_ref[0], xh_ref[1]], axis=1)
    h = part + coef * xh + b_ref[...]
    h = jnp.maximum(h, 0.0)
    xh2 = jnp.dot(h, w2_ref[...], preferred_element_type=jnp.float32)
    xh2_ref[0] = xh2[:, :FH]
    xh2_ref[1] = xh2[:, FH:]
    as2_ref[...] = jnp.sum(xh2 * asrc2_ref[...], axis=1, keepdims=True)
    ad2_ref[...] = jnp.sum(xh2 * adst2_ref[...], axis=1, keepdims=True)
    c2 = jnp.sum(we2_ref[...] * aedge2_ref[...])
    mean_w = scal_ref[2:3, :]
    ones = jnp.ones((1, LK), jnp.float32)
    scal2_ref[0:1, :] = ones * c2
    scal2_ref[1:2, :] = c2 * mean_w
    scal2_ref[2:3, :] = mean_w
    scal2_ref[3:4, :] = ones * 0.0


def _tc_post_body(part_ref, asum_ref, as_ref, ad_ref, scal_ref, xh_ref, b_ref,
                  out_ref):
    coef = _loop_coef(as_ref, ad_ref, asum_ref, scal_ref)
    part = jnp.concatenate([part_ref[0], part_ref[1]], axis=1)
    xh = jnp.concatenate([xh_ref[0], xh_ref[1]], axis=1)
    out_ref[...] = part + coef * xh + b_ref[...]


def _f32(shape):
    return jax.ShapeDtypeStruct(shape, jnp.float32)


_tc_pre = pl.pallas_call(
    _tc_pre_body,
    out_shape=(_f32((NC, N, FH)), _f32((N, 1)), _f32((N, 1)), _f32((4, LK))),
)

BN = 2000

_bs_split = pl.BlockSpec((NC, BN, FH), lambda i: (0, i, 0))
_bs_col = pl.BlockSpec((BN, 1), lambda i: (i, 0))
_bs_scal = pl.BlockSpec((4, LK), lambda i: (0, 0))
_bs_b = pl.BlockSpec((1, F), lambda i: (0, 0))
_bs_w = pl.BlockSpec((F, F), lambda i: (0, 0))

_tc_mid = pl.pallas_call(
    _tc_mid_body,
    grid=(N // BN,),
    in_specs=[_bs_split, _bs_col, _bs_col, _bs_col, _bs_scal, _bs_split,
              _bs_b, _bs_w, _bs_b, _bs_b, _bs_b, _bs_b],
    out_specs=(_bs_split, _bs_col, _bs_col, _bs_scal),
    out_shape=(_f32((NC, N, FH)), _f32((N, 1)), _f32((N, 1)), _f32((4, LK))),
)

_tc_post = pl.pallas_call(
    _tc_post_body,
    grid=(N // BN,),
    in_specs=[_bs_split, _bs_col, _bs_col, _bs_col, _bs_scal, _bs_split,
              _bs_b],
    out_specs=pl.BlockSpec((BN, F), lambda i: (i, 0)),
    out_shape=_f32((N, F)),
)


def _sc_body(xh_hbm, src_hbm, dst_hbm, ew_hbm, as_hbm, ad_hbm, scal_hbm,
             part_hbm, asum_hbm,
             src_w, dst_w, ew_w, as_v, ad_v, asum_v, rows_v,
             z640_v, st640_v, scal_v, out_sh, asum_sh, sem):
    c = lax.axis_index("c")
    s = lax.axis_index("s")
    zf = jnp.zeros((LK,), jnp.float32)

    pltpu.sync_copy(scal_hbm, scal_v)

    def _zrow(r, carry):
        for bb in range(2):
            for j in range(FH // LK):
                rows_v[bb, r, pl.ds(LK * j, LK)] = zf
        return carry
    lax.fori_loop(0, ROW, _zrow, 0)

    def _zz(i, carry):
        z640_v[pl.ds(LK * i, LK)] = zf
        return carry
    lax.fori_loop(0, 40, _zz, 0)

    base = s * 640
    tail_n = N - 640 * (NS - 1)

    def _zero_shared(nn):
        for t in range(nn // ROW):
            pltpu.sync_copy(rows_v.at[0], out_sh.at[pl.ds(base + ROW * t, ROW)])
        pltpu.sync_copy(z640_v.at[pl.ds(0, nn)], asum_sh.at[pl.ds(base, nn)])

    @pl.when(s < NS - 1)
    def _():
        _zero_shared(640)

    @pl.when(s == NS - 1)
    def _():
        _zero_shared(tail_n)

    plsc.subcore_barrier()

    c1v = scal_v[0, :]
    cm1v = scal_v[1, :]
    pltpu.sync_copy(as_hbm, as_v)
    pltpu.sync_copy(ad_hbm, ad_v)

    def _asca(r):
        return pltpu.make_async_copy(
            ew_w.at[r], asum_sh.at[dst_w.at[r]], sem.at[0])

    def _ph1_win(t, carry):
        pltpu.sync_copy(src_hbm.at[s, t], src_w)
        pltpu.sync_copy(dst_hbm.at[s, t], dst_w)
        pltpu.sync_copy(ew_hbm.at[s, t], ew_w)

        def _ph1_row(r, carry2):
            for k in range(ROW // LK):
                sl = pl.ds(LK * k, LK)
                s16 = src_w[r, sl]
                d16 = dst_w[r, sl]
                e16 = ew_w[r, sl]
                av = plsc.load_gather(as_v, [s16])
                bv = plsc.load_gather(ad_v, [d16])
                l = av + bv + c1v * e16
                l = jnp.where(l > 0, l, l * NEG_SLOPE)
                ew_w[r, sl] = jnp.exp(l)
            _asca(r).start(add=True)
            return carry2
        lax.fori_loop(0, WR, _ph1_row, 0)

        def _ph1_drain(r, carry2):
            _asca(r).wait()
            return carry2
        lax.fori_loop(0, WR, _ph1_drain, 0)
        return carry
    lax.fori_loop(0, NWIN, _ph1_win, 0)

    plsc.subcore_barrier()

    def _self_loop(nn):
        def _sl_chunk(i, carry):
            sl = pl.ds(LK * i, LK)
            l = (as_v[pl.ds(base + LK * i, LK)]
                 + ad_v[pl.ds(base + LK * i, LK)] + cm1v)
            l = jnp.where(l > 0, l, l * NEG_SLOPE)
            z640_v[sl] = jnp.exp(l)
            return carry
        lax.fori_loop(0, nn // LK, _sl_chunk, 0)
        pltpu.sync_copy(asum_sh.at[pl.ds(base, nn)], st640_v.at[pl.ds(0, nn)])

        def _sl_add(i, carry):
            sl = pl.ds(LK * i, LK)
            st640_v[sl] = st640_v[sl] + z640_v[sl]
            return carry
        lax.fori_loop(0, nn // LK, _sl_add, 0)
        pltpu.sync_copy(st640_v.at[pl.ds(0, nn)], asum_sh.at[pl.ds(base, nn)])

    @pl.when(s < NS - 1)
    def _():
        _self_loop(640)

    @pl.when(s == NS - 1)
    def _():
        _self_loop(tail_n)

    plsc.subcore_barrier()

    pltpu.sync_copy(asum_sh, asum_v)

    @pl.when(jnp.logical_and(c == 0, s < NS - 1))
    def _():
        pltpu.sync_copy(asum_v.at[pl.ds(base, 640)],
                        asum_hbm.at[pl.ds(base, 640)])

    @pl.when(jnp.logical_and(c == 0, s == NS - 1))
    def _():
        pltpu.sync_copy(asum_v.at[pl.ds(base, tail_n)],
                        asum_hbm.at[pl.ds(base, tail_n)])

    def _gat(r, b):
        return pltpu.make_async_copy(
            xh_hbm.at[c].at[src_w.at[r]], rows_v.at[b], sem.at[b])

    def _sca(r, b):
        return pltpu.make_async_copy(
            rows_v.at[b], out_sh.at[dst_w.at[r]], sem.at[2 + b])

    def _ph2_win(t, carry):
        pltpu.sync_copy(src_hbm.at[s, t], src_w)
        pltpu.sync_copy(dst_hbm.at[s, t], dst_w)
        pltpu.sync_copy(ew_hbm.at[s, t], ew_w)
        _gat(0, 0).start()

        def _ph2_row(r, carry2):
            b = r % 2
            _gat(r, b).wait()

            @pl.when(r >= 1)
            def _():
                _sca(r - 1, 1 - b).wait()

            @pl.when(r < WR - 1)
            def _():
                _gat(r + 1, 1 - b).start()

            for k16 in range(ROW // LK):
                sl = pl.ds(LK * k16, LK)
                s16 = src_w[r, sl]
                d16 = dst_w[r, sl]
                e16 = ew_w[r, sl]
                av = plsc.load_gather(as_v, [s16])
                bv = plsc.load_gather(ad_v, [d16])
                l = av + bv + c1v * e16
                l = jnp.where(l > 0, l, l * NEG_SLOPE)
                p16 = jnp.exp(l)
                sa = plsc.load_gather(asum_v, [d16])
                coefc = p16 / (sa + EPS)
                for kk in range(LK):
                    cb = _bcast16(coefc, kk)
                    k = LK * k16 + kk
                    for j8 in range(FH // LK):
                        sl2 = pl.ds(LK * j8, LK)
                        rows_v[b, k, sl2] = rows_v[b, k, sl2] * cb
            _sca(r, b).start(add=True)
            return carry2
        lax.fori_loop(0, WR, _ph2_row, 0)
        _sca(WR - 1, (WR - 1) % 2).wait()
        return carry
    lax.fori_loop(0, NWIN, _ph2_win, 0)

    plsc.subcore_barrier()

    def _wout(nn):
        for t in range(nn // ROW):
            bb = t % 2
            pltpu.sync_copy(out_sh.at[pl.ds(base + ROW * t, ROW)], rows_v.at[bb])
            pltpu.sync_copy(rows_v.at[bb], part_hbm.at[c, pl.ds(base + ROW * t, ROW)])

    @pl.when(s < NS - 1)
    def _():
        _wout(640)

    @pl.when(s == NS - 1)
    def _():
        _wout(tail_n)


_sc_edge_pass = pl.kernel(
    _sc_body,
    out_type=(_f32((NC, N, FH)), _f32((N,))),
    mesh=plsc.VectorSubcoreMesh(core_axis_name="c", subcore_axis_name="s"),
    compiler_params=pltpu.CompilerParams(
        needs_layout_passes=False, use_tc_tiling_on_sc=False),
    scratch_types=[
        pltpu.VMEM((WR, ROW), jnp.int32),
        pltpu.VMEM((WR, ROW), jnp.int32),
        pltpu.VMEM((WR, ROW), jnp.float32),
        pltpu.VMEM((N,), jnp.float32),
        pltpu.VMEM((N,), jnp.float32),
        pltpu.VMEM((N,), jnp.float32),
        pltpu.VMEM((2, ROW, FH), jnp.float32),
        pltpu.VMEM((640,), jnp.float32),
        pltpu.VMEM((640,), jnp.float32),
        pltpu.VMEM((4, LK), jnp.float32),
        pltpu.VMEM_SHARED((N, FH), jnp.float32),
        pltpu.VMEM_SHARED((N,), jnp.float32),
        pltpu.SemaphoreType.DMA((4,)),
    ],
)


def kernel(x, edge_index, edge_weight, W1, a_src1, a_dst1, a_edge1, We1, b1,
           W2, a_src2, a_dst2, a_edge2, We2, b2):
    src2d = edge_index[0].reshape(NS, NWIN, WR, ROW)
    dst2d = edge_index[1].reshape(NS, NWIN, WR, ROW)
    ew2d = edge_weight.reshape(NS, NWIN, WR, ROW)

    xh1, as1, ad1, scal1 = _tc_pre(
        x, W1, a_src1.reshape(1, F), a_dst1.reshape(1, F),
        a_edge1.reshape(1, F), We1.reshape(1, F), edge_weight.reshape(-1, F))
    part1, asum1 = _sc_edge_pass(
        xh1, src2d, dst2d, ew2d, as1.reshape(N), ad1.reshape(N), scal1)
    xh2, as2, ad2, scal2 = _tc_mid(
        part1, asum1.reshape(N, 1), as1, ad1, scal1, xh1, b1.reshape(1, F),
        W2, a_src2.reshape(1, F), a_dst2.reshape(1, F),
        a_edge2.reshape(1, F), We2.reshape(1, F))
    part2, asum2 = _sc_edge_pass(
        xh2, src2d, dst2d, ew2d, as2.reshape(N), ad2.reshape(N), scal2)
    return _tc_post(
        part2, asum2.reshape(N, 1), as2, ad2, scal2, xh2, b2.reshape(1, F))


# speedup vs baseline: 32.8630x; 1.1351x over previous
import functools

import jax
import jax.numpy as jnp
from jax import lax
from jax.experimental import pallas as pl
from jax.experimental.pallas import tpu as pltpu
from jax.experimental.pallas import tpu_sc as plsc

N = 10000
E = 320000
F = 128
NC, NS = 2, 16
ROW = 80
LK = 16
NWIN = 10
WR = 25
FH = F // NC
NEG_SLOPE = 0.2
EPS = 1e-16

_GDN = lax.GatherDimensionNumbers(
    offset_dims=(), collapsed_slice_dims=(0,), start_index_map=(0,))


def _bcast16(vec, k):
    idx = jnp.full((LK, 1), k, jnp.int32)
    return lax.gather(vec, idx, _GDN, (1,),
                      mode=lax.GatherScatterMode.PROMISE_IN_BOUNDS)


def _tc_pre_body(x_ref, w_ref, asrc_ref, adst_ref, aedge_ref, we_ref, ew_ref,
                 xh_ref, as_ref, ad_ref, scal_ref):
    xh = jnp.dot(x_ref[...], w_ref[...], preferred_element_type=jnp.float32)
    xh_ref[0] = xh[:, :FH]
    xh_ref[1] = xh[:, FH:]
    as_ref[...] = jnp.sum(xh * asrc_ref[...], axis=1, keepdims=True)
    ad_ref[...] = jnp.sum(xh * adst_ref[...], axis=1, keepdims=True)
    c = jnp.sum(we_ref[...] * aedge_ref[...])
    mean_w = jnp.mean(ew_ref[...])
    ones = jnp.ones((1, LK), jnp.float32)
    scal_ref[0:1, :] = ones * c
    scal_ref[1:2, :] = ones * (c * mean_w)
    scal_ref[2:3, :] = ones * mean_w
    scal_ref[3:4, :] = ones * 0.0


def _loop_coef(as_ref, ad_ref, asum_ref, scal_ref):
    l = as_ref[...] + ad_ref[...] + scal_ref[1:2, 0:1]
    l = jnp.where(l > 0, l, l * NEG_SLOPE)
    p = jnp.exp(l)
    inv = 1.0 / (asum_ref[...] + p + EPS)
    return p * inv, inv


def _tc_mid_body(part_ref, asum_ref, as_ref, ad_ref, scal_ref, xh_ref, b_ref,
                 w2_ref, asrc2_ref, adst2_ref, aedge2_ref, we2_ref,
                 xh2_ref, as2_ref, ad2_ref, scal2_ref):
    coef, inv = _loop_coef(as_ref, ad_ref, asum_ref, scal_ref)
    part = jnp.concatenate([part_ref[0], part_ref[1]], axis=1)
    xh = jnp.concatenate([xh_ref[0], xh_ref[1]], axis=1)
    h = part * inv + coef * xh + b_ref[...]
    h = jnp.maximum(h, 0.0)
    xh2 = jnp.dot(h, w2_ref[...], preferred_element_type=jnp.float32)
    xh2_ref[0] = xh2[:, :FH]
    xh2_ref[1] = xh2[:, FH:]
    as2_ref[...] = jnp.sum(xh2 * asrc2_ref[...], axis=1, keepdims=True)
    ad2_ref[...] = jnp.sum(xh2 * adst2_ref[...], axis=1, keepdims=True)
    c2 = jnp.sum(we2_ref[...] * aedge2_ref[...])
    mean_w = scal_ref[2:3, :]
    ones = jnp.ones((1, LK), jnp.float32)
    scal2_ref[0:1, :] = ones * c2
    scal2_ref[1:2, :] = c2 * mean_w
    scal2_ref[2:3, :] = mean_w
    scal2_ref[3:4, :] = ones * 0.0


def _tc_post_body(part_ref, asum_ref, as_ref, ad_ref, scal_ref, xh_ref, b_ref,
                  out_ref):
    coef, inv = _loop_coef(as_ref, ad_ref, asum_ref, scal_ref)
    part = jnp.concatenate([part_ref[0], part_ref[1]], axis=1)
    xh = jnp.concatenate([xh_ref[0], xh_ref[1]], axis=1)
    out_ref[...] = part * inv + coef * xh + b_ref[...]


def _f32(shape):
    return jax.ShapeDtypeStruct(shape, jnp.float32)


_tc_pre = pl.pallas_call(
    _tc_pre_body,
    out_shape=(_f32((NC, N, FH)), _f32((N, 1)), _f32((N, 1)), _f32((4, LK))),
)

BN = 2000

_bs_split = pl.BlockSpec((NC, BN, FH), lambda i: (0, i, 0))
_bs_col = pl.BlockSpec((BN, 1), lambda i: (i, 0))
_bs_scal = pl.BlockSpec((4, LK), lambda i: (0, 0))
_bs_b = pl.BlockSpec((1, F), lambda i: (0, 0))
_bs_w = pl.BlockSpec((F, F), lambda i: (0, 0))

_tc_mid = pl.pallas_call(
    _tc_mid_body,
    grid=(N // BN,),
    in_specs=[_bs_split, _bs_col, _bs_col, _bs_col, _bs_scal, _bs_split,
              _bs_b, _bs_w, _bs_b, _bs_b, _bs_b, _bs_b],
    out_specs=(_bs_split, _bs_col, _bs_col, _bs_scal),
    out_shape=(_f32((NC, N, FH)), _f32((N, 1)), _f32((N, 1)), _f32((4, LK))),
)

_tc_post = pl.pallas_call(
    _tc_post_body,
    grid=(N // BN,),
    in_specs=[_bs_split, _bs_col, _bs_col, _bs_col, _bs_scal, _bs_split,
              _bs_b],
    out_specs=pl.BlockSpec((BN, F), lambda i: (i, 0)),
    out_shape=_f32((N, F)),
)


def _sc_body(xh_hbm, src_hbm, dst_hbm, ew_hbm, as_hbm, ad_hbm, scal_hbm,
             part_hbm, asum_hbm,
             src_w, dst_w, ew_w, as_v, ad_v, rows_v, z640_v, scal_v,
             out_sh, asum_sh, sem):
    c = lax.axis_index("c")
    s = lax.axis_index("s")
    zf = jnp.zeros((LK,), jnp.float32)

    pltpu.sync_copy(scal_hbm, scal_v)

    def _zrow(r, carry):
        for j in range(FH // LK):
            rows_v[0, r, pl.ds(LK * j, LK)] = zf
        return carry
    lax.fori_loop(0, ROW, _zrow, 0)

    def _zz(i, carry):
        z640_v[pl.ds(LK * i, LK)] = zf
        return carry
    lax.fori_loop(0, 40, _zz, 0)

    base = s * 640
    tail_n = N - 640 * (NS - 1)

    def _zero_shared(nn):
        for t in range(nn // ROW):
            pltpu.sync_copy(rows_v.at[0], out_sh.at[pl.ds(base + ROW * t, ROW)])
        pltpu.sync_copy(z640_v.at[pl.ds(0, nn)], asum_sh.at[pl.ds(base, nn)])

    @pl.when(s < NS - 1)
    def _():
        _zero_shared(640)

    @pl.when(s == NS - 1)
    def _():
        _zero_shared(tail_n)

    plsc.subcore_barrier()

    c1v = scal_v[0, :]
    pltpu.sync_copy(as_hbm, as_v)
    pltpu.sync_copy(ad_hbm, ad_v)

    def _asca(r):
        return pltpu.make_async_copy(
            ew_w.at[r], asum_sh.at[dst_w.at[r]], sem.at[4])

    def _gat(r, b):
        return pltpu.make_async_copy(
            xh_hbm.at[c].at[src_w.at[r]], rows_v.at[b], sem.at[b])

    def _sca(r, b):
        return pltpu.make_async_copy(
            rows_v.at[b], out_sh.at[dst_w.at[r]], sem.at[2 + b])

    def _win(t, carry):
        pltpu.sync_copy(src_hbm.at[s, t], src_w)
        pltpu.sync_copy(dst_hbm.at[s, t], dst_w)
        pltpu.sync_copy(ew_hbm.at[s, t], ew_w)
        _gat(0, 0).start()

        def _row(r, carry2):
            b = r % 2
            for k in range(ROW // LK):
                sl = pl.ds(LK * k, LK)
                s16 = src_w[r, sl]
                d16 = dst_w[r, sl]
                e16 = ew_w[r, sl]
                av = plsc.load_gather(as_v, [s16])
                bv = plsc.load_gather(ad_v, [d16])
                l = av + bv + c1v * e16
                l = jnp.where(l > 0, l, l * NEG_SLOPE)
                ew_w[r, sl] = jnp.exp(l)
            _asca(r).start(add=True)
            _gat(r, b).wait()

            @pl.when(r >= 1)
            def _():
                _sca(r - 1, 1 - b).wait()

            @pl.when(r < WR - 1)
            def _():
                _gat(r + 1, 1 - b).start()

            for k16 in range(ROW // LK):
                sl = pl.ds(LK * k16, LK)
                p16 = ew_w[r, sl]
                for kk in range(LK):
                    cb = _bcast16(p16, kk)
                    k = LK * k16 + kk
                    for j8 in range(FH // LK):
                        sl2 = pl.ds(LK * j8, LK)
                        rows_v[b, k, sl2] = rows_v[b, k, sl2] * cb
            _sca(r, b).start(add=True)
            return carry2
        lax.fori_loop(0, WR, _row, 0)
        _sca(WR - 1, (WR - 1) % 2).wait()

        def _drain(r, carry2):
            _asca(r).wait()
            return carry2
        lax.fori_loop(0, WR, _drain, 0)
        return carry
    lax.fori_loop(0, NWIN, _win, 0)

    plsc.subcore_barrier()

    def _wout(nn):
        for t in range(nn // ROW):
            bb = t % 2
            pltpu.sync_copy(out_sh.at[pl.ds(base + ROW * t, ROW)], rows_v.at[bb])
            pltpu.sync_copy(rows_v.at[bb], part_hbm.at[c, pl.ds(base + ROW * t, ROW)])

    @pl.when(s < NS - 1)
    def _():
        _wout(640)

    @pl.when(s == NS - 1)
    def _():
        _wout(tail_n)

    @pl.when(jnp.logical_and(c == 0, s < NS - 1))
    def _():
        pltpu.sync_copy(asum_sh.at[pl.ds(base, 640)], z640_v.at[pl.ds(0, 640)])
        pltpu.sync_copy(z640_v.at[pl.ds(0, 640)], asum_hbm.at[pl.ds(base, 640)])

    @pl.when(jnp.logical_and(c == 0, s == NS - 1))
    def _():
        pltpu.sync_copy(asum_sh.at[pl.ds(base, tail_n)],
                        z640_v.at[pl.ds(0, tail_n)])
        pltpu.sync_copy(z640_v.at[pl.ds(0, tail_n)],
                        asum_hbm.at[pl.ds(base, tail_n)])


_sc_edge_pass = pl.kernel(
    _sc_body,
    out_type=(_f32((NC, N, FH)), _f32((N,))),
    mesh=plsc.VectorSubcoreMesh(core_axis_name="c", subcore_axis_name="s"),
    compiler_params=pltpu.CompilerParams(
        needs_layout_passes=False, use_tc_tiling_on_sc=False),
    scratch_types=[
        pltpu.VMEM((WR, ROW), jnp.int32),
        pltpu.VMEM((WR, ROW), jnp.int32),
        pltpu.VMEM((WR, ROW), jnp.float32),
        pltpu.VMEM((N,), jnp.float32),
        pltpu.VMEM((N,), jnp.float32),
        pltpu.VMEM((2, ROW, FH), jnp.float32),
        pltpu.VMEM((640,), jnp.float32),
        pltpu.VMEM((4, LK), jnp.float32),
        pltpu.VMEM_SHARED((N, FH), jnp.float32),
        pltpu.VMEM_SHARED((N,), jnp.float32),
        pltpu.SemaphoreType.DMA((5,)),
    ],
)


def kernel(x, edge_index, edge_weight, W1, a_src1, a_dst1, a_edge1, We1, b1,
           W2, a_src2, a_dst2, a_edge2, We2, b2):
    src2d = edge_index[0].reshape(NS, NWIN, WR, ROW)
    dst2d = edge_index[1].reshape(NS, NWIN, WR, ROW)
    ew2d = edge_weight.reshape(NS, NWIN, WR, ROW)

    xh1, as1, ad1, scal1 = _tc_pre(
        x, W1, a_src1.reshape(1, F), a_dst1.reshape(1, F),
        a_edge1.reshape(1, F), We1.reshape(1, F), edge_weight.reshape(-1, F))
    part1, asum1 = _sc_edge_pass(
        xh1, src2d, dst2d, ew2d, as1.reshape(N), ad1.reshape(N), scal1)
    xh2, as2, ad2, scal2 = _tc_mid(
        part1, asum1.reshape(N, 1), as1, ad1, scal1, xh1, b1.reshape(1, F),
        W2, a_src2.reshape(1, F), a_dst2.reshape(1, F),
        a_edge2.reshape(1, F), We2.reshape(1, F))
    part2, asum2 = _sc_edge_pass(
        xh2, src2d, dst2d, ew2d, as2.reshape(N), ad2.reshape(N), scal2)
    return _tc_post(
        part2, asum2.reshape(N, 1), as2, ad2, scal2, xh2, b2.reshape(1, F))


# speedup vs baseline: 34.2368x; 1.0418x over previous
import functools

import jax
import jax.numpy as jnp
from jax import lax
from jax.experimental import pallas as pl
from jax.experimental.pallas import tpu as pltpu
from jax.experimental.pallas import tpu_sc as plsc

N = 10000
E = 320000
F = 128
NC, NS = 2, 16
ROW = 80
LK = 16
NWIN = 10
WR = 25
FH = F // NC
NEG_SLOPE = 0.2
EPS = 1e-16

_GDN = lax.GatherDimensionNumbers(
    offset_dims=(), collapsed_slice_dims=(0,), start_index_map=(0,))


def _bcast16(vec, k):
    idx = jnp.full((LK, 1), k, jnp.int32)
    return lax.gather(vec, idx, _GDN, (1,),
                      mode=lax.GatherScatterMode.PROMISE_IN_BOUNDS)


def _tc_pre_body(x_ref, w_ref, asrc_ref, adst_ref, aedge_ref, we_ref, ew_ref,
                 xh_ref, as_ref, ad_ref, scal_ref):
    xh = jnp.dot(x_ref[...], w_ref[...], preferred_element_type=jnp.float32)
    xh_ref[0] = xh[:, :FH]
    xh_ref[1] = xh[:, FH:]
    as_ref[...] = jnp.sum(xh * asrc_ref[...], axis=1, keepdims=True)
    ad_ref[...] = jnp.sum(xh * adst_ref[...], axis=1, keepdims=True)
    c = jnp.sum(we_ref[...] * aedge_ref[...])
    mean_w = jnp.mean(ew_ref[...])
    ones = jnp.ones((1, LK), jnp.float32)
    scal_ref[0:1, :] = ones * c
    scal_ref[1:2, :] = ones * (c * mean_w)
    scal_ref[2:3, :] = ones * mean_w
    scal_ref[3:4, :] = ones * 0.0


def _loop_coef(as_ref, ad_ref, asum_ref, scal_ref):
    l = as_ref[...] + ad_ref[...] + scal_ref[1:2, 0:1]
    l = jnp.where(l > 0, l, l * NEG_SLOPE)
    p = jnp.exp(l)
    inv = 1.0 / (asum_ref[...] + p + EPS)
    return p * inv, inv


def _tc_mid_body(part_ref, asum_ref, as_ref, ad_ref, scal_ref, xh_ref, b_ref,
                 w2_ref, asrc2_ref, adst2_ref, aedge2_ref, we2_ref,
                 xh2_ref, as2_ref, ad2_ref, scal2_ref):
    coef, inv = _loop_coef(as_ref, ad_ref, asum_ref, scal_ref)
    part = jnp.concatenate([part_ref[0], part_ref[1]], axis=1)
    xh = jnp.concatenate([xh_ref[0], xh_ref[1]], axis=1)
    h = part * inv + coef * xh + b_ref[...]
    h = jnp.maximum(h, 0.0)
    xh2 = jnp.dot(h, w2_ref[...], preferred_element_type=jnp.float32)
    xh2_ref[0] = xh2[:, :FH]
    xh2_ref[1] = xh2[:, FH:]
    as2_ref[...] = jnp.sum(xh2 * asrc2_ref[...], axis=1, keepdims=True)
    ad2_ref[...] = jnp.sum(xh2 * adst2_ref[...], axis=1, keepdims=True)
    c2 = jnp.sum(we2_ref[...] * aedge2_ref[...])
    mean_w = scal_ref[2:3, :]
    ones = jnp.ones((1, LK), jnp.float32)
    scal2_ref[0:1, :] = ones * c2
    scal2_ref[1:2, :] = c2 * mean_w
    scal2_ref[2:3, :] = mean_w
    scal2_ref[3:4, :] = ones * 0.0


def _tc_post_body(part_ref, asum_ref, as_ref, ad_ref, scal_ref, xh_ref, b_ref,
                  out_ref):
    coef, inv = _loop_coef(as_ref, ad_ref, asum_ref, scal_ref)
    part = jnp.concatenate([part_ref[0], part_ref[1]], axis=1)
    xh = jnp.concatenate([xh_ref[0], xh_ref[1]], axis=1)
    out_ref[...] = part * inv + coef * xh + b_ref[...]


def _f32(shape):
    return jax.ShapeDtypeStruct(shape, jnp.float32)


_tc_pre = pl.pallas_call(
    _tc_pre_body,
    out_shape=(_f32((NC, N, FH)), _f32((N, 1)), _f32((N, 1)), _f32((4, LK))),
)

BN = 2000

_bs_split = pl.BlockSpec((NC, BN, FH), lambda i: (0, i, 0))
_bs_col = pl.BlockSpec((BN, 1), lambda i: (i, 0))
_bs_scal = pl.BlockSpec((4, LK), lambda i: (0, 0))
_bs_b = pl.BlockSpec((1, F), lambda i: (0, 0))
_bs_w = pl.BlockSpec((F, F), lambda i: (0, 0))

_tc_mid = pl.pallas_call(
    _tc_mid_body,
    grid=(N // BN,),
    in_specs=[_bs_split, _bs_col, _bs_col, _bs_col, _bs_scal, _bs_split,
              _bs_b, _bs_w, _bs_b, _bs_b, _bs_b, _bs_b],
    out_specs=(_bs_split, _bs_col, _bs_col, _bs_scal),
    out_shape=(_f32((NC, N, FH)), _f32((N, 1)), _f32((N, 1)), _f32((4, LK))),
)

_tc_post = pl.pallas_call(
    _tc_post_body,
    grid=(N // BN,),
    in_specs=[_bs_split, _bs_col, _bs_col, _bs_col, _bs_scal, _bs_split,
              _bs_b],
    out_specs=pl.BlockSpec((BN, F), lambda i: (i, 0)),
    out_shape=_f32((N, F)),
)


def _sc_body(xh_hbm, src_hbm, dst_hbm, ew_hbm, as_hbm, ad_hbm, scal_hbm,
             part_hbm, asum_hbm,
             src_w0, src_w1, dst_w0, dst_w1, ew_w0, ew_w1,
             as_v, ad_v, rows_v, z640_v, scal_v,
             out_sh, asum_sh, sem):
    c = lax.axis_index("c")
    s = lax.axis_index("s")
    zf = jnp.zeros((LK,), jnp.float32)

    pltpu.sync_copy(scal_hbm, scal_v)

    def _zrow(r, carry):
        for j in range(FH // LK):
            rows_v[0, r, pl.ds(LK * j, LK)] = zf
        return carry
    lax.fori_loop(0, ROW, _zrow, 0)

    def _zz(i, carry):
        z640_v[pl.ds(LK * i, LK)] = zf
        return carry
    lax.fori_loop(0, 40, _zz, 0)

    base = s * 640
    tail_n = N - 640 * (NS - 1)

    def _zero_shared(nn):
        for t in range(nn // ROW):
            pltpu.sync_copy(rows_v.at[0], out_sh.at[pl.ds(base + ROW * t, ROW)])
        pltpu.sync_copy(z640_v.at[pl.ds(0, nn)], asum_sh.at[pl.ds(base, nn)])

    @pl.when(s < NS - 1)
    def _():
        _zero_shared(640)

    @pl.when(s == NS - 1)
    def _():
        _zero_shared(tail_n)

    c1v = scal_v[0, :]

    def _asca(r, ewb, dwb):
        return pltpu.make_async_copy(
            ewb.at[r], asum_sh.at[dwb.at[r]], sem.at[4])

    def _gat(r, b, swb):
        return pltpu.make_async_copy(
            xh_hbm.at[c].at[swb.at[r]], rows_v.at[b], sem.at[b])

    def _sca(r, b, dwb):
        return pltpu.make_async_copy(
            rows_v.at[b], out_sh.at[dwb.at[r]], sem.at[2 + b])

    def _wcopies(t, swb, dwb, ewb):
        return (pltpu.make_async_copy(src_hbm.at[s, t], swb, sem.at[5]),
                pltpu.make_async_copy(dst_hbm.at[s, t], dwb, sem.at[6]),
                pltpu.make_async_copy(ew_hbm.at[s, t], ewb, sem.at[7]))

    bufs = ((src_w0, dst_w0, ew_w0), (src_w1, dst_w1, ew_w1))
    for cp in _wcopies(0, *bufs[0]):
        cp.start()
    pltpu.make_async_copy(as_hbm, as_v, sem.at[8]).start()
    pltpu.make_async_copy(ad_hbm, ad_v, sem.at[9]).start()

    plsc.subcore_barrier()

    pltpu.make_async_copy(as_hbm, as_v, sem.at[8]).wait()
    pltpu.make_async_copy(ad_hbm, ad_v, sem.at[9]).wait()

    for t in range(NWIN):
        swb, dwb, ewb = bufs[t % 2]
        for cp in _wcopies(t, swb, dwb, ewb):
            cp.wait()
        if t < NWIN - 1:
            for cp in _wcopies(t + 1, *bufs[(t + 1) % 2]):
                cp.start()
        _gat(0, 0, swb).start()

        def _row(r, carry2, swb=swb, dwb=dwb, ewb=ewb):
            b = r % 2
            for k in range(ROW // LK):
                sl = pl.ds(LK * k, LK)
                s16 = swb[r, sl]
                d16 = dwb[r, sl]
                e16 = ewb[r, sl]
                av = plsc.load_gather(as_v, [s16])
                bv = plsc.load_gather(ad_v, [d16])
                l = av + bv + c1v * e16
                l = jnp.where(l > 0, l, l * NEG_SLOPE)
                ewb[r, sl] = jnp.exp(l)
            _asca(r, ewb, dwb).start(add=True)
            _gat(r, b, swb).wait()

            @pl.when(r >= 1)
            def _():
                _sca(r - 1, 1 - b, dwb).wait()

            @pl.when(r < WR - 1)
            def _():
                _gat(r + 1, 1 - b, swb).start()

            for k16 in range(ROW // LK):
                sl = pl.ds(LK * k16, LK)
                p16 = ewb[r, sl]
                for kk in range(LK):
                    cb = _bcast16(p16, kk)
                    k = LK * k16 + kk
                    for j8 in range(FH // LK):
                        sl2 = pl.ds(LK * j8, LK)
                        rows_v[b, k, sl2] = rows_v[b, k, sl2] * cb
            _sca(r, b, dwb).start(add=True)
            return carry2
        lax.fori_loop(0, WR, _row, 0)
        _sca(WR - 1, (WR - 1) % 2, dwb).wait()

        def _drain(r, carry2, ewb=ewb, dwb=dwb):
            _asca(r, ewb, dwb).wait()
            return carry2
        lax.fori_loop(0, WR, _drain, 0)

    plsc.subcore_barrier()

    def _wout(nn):
        for t in range(nn // ROW):
            bb = t % 2
            pltpu.sync_copy(out_sh.at[pl.ds(base + ROW * t, ROW)], rows_v.at[bb])
            pltpu.sync_copy(rows_v.at[bb], part_hbm.at[c, pl.ds(base + ROW * t, ROW)])

    @pl.when(s < NS - 1)
    def _():
        _wout(640)

    @pl.when(s == NS - 1)
    def _():
        _wout(tail_n)

    @pl.when(jnp.logical_and(c == 0, s < NS - 1))
    def _():
        pltpu.sync_copy(asum_sh.at[pl.ds(base, 640)], z640_v.at[pl.ds(0, 640)])
        pltpu.sync_copy(z640_v.at[pl.ds(0, 640)], asum_hbm.at[pl.ds(base, 640)])

    @pl.when(jnp.logical_and(c == 0, s == NS - 1))
    def _():
        pltpu.sync_copy(asum_sh.at[pl.ds(base, tail_n)],
                        z640_v.at[pl.ds(0, tail_n)])
        pltpu.sync_copy(z640_v.at[pl.ds(0, tail_n)],
                        asum_hbm.at[pl.ds(base, tail_n)])


_sc_edge_pass = pl.kernel(
    _sc_body,
    out_type=(_f32((NC, N, FH)), _f32((N,))),
    mesh=plsc.VectorSubcoreMesh(core_axis_name="c", subcore_axis_name="s"),
    compiler_params=pltpu.CompilerParams(
        needs_layout_passes=False, use_tc_tiling_on_sc=False),
    scratch_types=[
        pltpu.VMEM((WR, ROW), jnp.int32),
        pltpu.VMEM((WR, ROW), jnp.int32),
        pltpu.VMEM((WR, ROW), jnp.int32),
        pltpu.VMEM((WR, ROW), jnp.int32),
        pltpu.VMEM((WR, ROW), jnp.float32),
        pltpu.VMEM((WR, ROW), jnp.float32),
        pltpu.VMEM((N,), jnp.float32),
        pltpu.VMEM((N,), jnp.float32),
        pltpu.VMEM((2, ROW, FH), jnp.float32),
        pltpu.VMEM((640,), jnp.float32),
        pltpu.VMEM((4, LK), jnp.float32),
        pltpu.VMEM_SHARED((N, FH), jnp.float32),
        pltpu.VMEM_SHARED((N,), jnp.float32),
        pltpu.SemaphoreType.DMA((10,)),
    ],
)


def kernel(x, edge_index, edge_weight, W1, a_src1, a_dst1, a_edge1, We1, b1,
           W2, a_src2, a_dst2, a_edge2, We2, b2):
    src2d = edge_index[0].reshape(NS, NWIN, WR, ROW)
    dst2d = edge_index[1].reshape(NS, NWIN, WR, ROW)
    ew2d = edge_weight.reshape(NS, NWIN, WR, ROW)

    xh1, as1, ad1, scal1 = _tc_pre(
        x, W1, a_src1.reshape(1, F), a_dst1.reshape(1, F),
        a_edge1.reshape(1, F), We1.reshape(1, F), edge_weight.reshape(-1, F))
    part1, asum1 = _sc_edge_pass(
        xh1, src2d, dst2d, ew2d, as1.reshape(N), ad1.reshape(N), scal1)
    xh2, as2, ad2, scal2 = _tc_mid(
        part1, asum1.reshape(N, 1), as1, ad1, scal1, xh1, b1.reshape(1, F),
        W2, a_src2.reshape(1, F), a_dst2.reshape(1, F),
        a_edge2.reshape(1, F), We2.reshape(1, F))
    part2, asum2 = _sc_edge_pass(
        xh2, src2d, dst2d, ew2d, as2.reshape(N), ad2.reshape(N), scal2)
    return _tc_post(
        part2, asum2.reshape(N, 1), as2, ad2, scal2, xh2, b2.reshape(1, F))
